# Initial kernel scaffold; baseline (speedup 1.0000x reference)
import jax
import jax.numpy as jnp
from jax.experimental import pallas as pl


def kernel(x, edge_index, W1, b1, W2, b2, W3, b3):
    raise NotImplementedError("write your pallas kernel here")


# speedup vs baseline: 22.8368x; 22.8368x over previous
import functools

import jax
import jax.numpy as jnp
from jax import lax
from jax.experimental import pallas as pl
from jax.experimental.pallas import tpu as pltpu
from jax.experimental.pallas import tpu_sc as plsc

N = 10000
E = 320000
D_IN = 128
D_HID = 64

NC = 2
NS = 16
NW = NC * NS
K = 128
NB = -(-E // (NW * K))
E_PAD = NW * K * NB
N_ACC = 10112
RPS = N_ACC // NS

def _sc_mesh():
    return plsc.VectorSubcoreMesh(core_axis_name="c", subcore_axis_name="s",
                                  num_cores=NC, num_subcores=NS)


@functools.lru_cache(maxsize=None)
def _make_edge_agg(d):
    @functools.partial(
        pl.kernel,
        out_type=jax.ShapeDtypeStruct((NC, N_ACC, d), jnp.float32),
        mesh=_sc_mesh(),
        compiler_params=pltpu.CompilerParams(use_tc_tiling_on_sc=False),
        scratch_types=[
            pltpu.VMEM((NB, K), jnp.int32),
            pltpu.VMEM((NB, K), jnp.int32),
            pltpu.VMEM((K, d), jnp.float32),
            pltpu.VMEM_SHARED((N_ACC, d), jnp.float32),
            pltpu.SemaphoreType.DMA,
        ],
    )
    def agg(p_hbm, src_hbm, dst_hbm, zeros_hbm, out_hbm,
            src_v, dst_v, rows_v, acc_sh, sem):
        cid = lax.axis_index("c")
        sid = lax.axis_index("s")
        wid = sid * NC + cid
        pltpu.sync_copy(src_hbm.at[wid], src_v)
        pltpu.sync_copy(dst_hbm.at[wid], dst_v)
        pltpu.sync_copy(zeros_hbm, acc_sh.at[pl.ds(sid * RPS, RPS)])
        plsc.subcore_barrier()

        def body(j, carry):
            pltpu.async_copy(p_hbm.at[src_v.at[j]], rows_v, sem).wait()
            pltpu.sync_copy(rows_v, acc_sh.at[dst_v.at[j]], add=True)
            return carry

        lax.fori_loop(0, NB, body, 0)
        plsc.subcore_barrier()
        pltpu.sync_copy(acc_sh.at[pl.ds(sid * RPS, RPS)],
                        out_hbm.at[cid, pl.ds(sid * RPS, RPS)])

    return agg


@functools.lru_cache(maxsize=None)
def _make_deg():
    @functools.partial(
        pl.kernel,
        out_type=jax.ShapeDtypeStruct((NC, N_ACC, 16), jnp.float32),
        mesh=_sc_mesh(),
        compiler_params=pltpu.CompilerParams(use_tc_tiling_on_sc=False),
        scratch_types=[
            pltpu.VMEM((NB, K), jnp.int32),
            pltpu.VMEM((K, 16), jnp.float32),
            pltpu.VMEM_SHARED((N_ACC, 16), jnp.float32),
        ],
    )
    def deg(dst_hbm, ones_hbm, zeros_hbm, out_hbm, dst_v, ones_v, acc_sh):
        cid = lax.axis_index("c")
        sid = lax.axis_index("s")
        wid = sid * NC + cid
        pltpu.sync_copy(dst_hbm.at[wid], dst_v)
        pltpu.sync_copy(ones_hbm, ones_v)
        pltpu.sync_copy(zeros_hbm, acc_sh.at[pl.ds(sid * RPS, RPS)])
        plsc.subcore_barrier()

        def body(j, carry):
            pltpu.sync_copy(ones_v, acc_sh.at[dst_v.at[j]], add=True)
            return carry

        lax.fori_loop(0, NB, body, 0)
        plsc.subcore_barrier()
        pltpu.sync_copy(acc_sh.at[pl.ds(sid * RPS, RPS)],
                        out_hbm.at[cid, pl.ds(sid * RPS, RPS)])

    return deg


_R = 1000


def _dinv_of(d0, d1):
    return lax.rsqrt(d0[:, 0:1] + d1[:, 0:1] + 1.0)


def _stage_a_body(x_ref, w1_ref, d0_ref, d1_ref, p1_ref):
    dinv = _dinv_of(d0_ref[...], d1_ref[...])
    h0 = lax.dot_general(x_ref[...], w1_ref[...], (((1,), (1,)), ((), ())),
                         preferred_element_type=jnp.float32)
    p1_ref[...] = h0 * dinv


def _stage_b_body(s0_ref, s1_ref, p1_ref, d0_ref, d1_ref, b1_ref, p2_ref):
    dinv = _dinv_of(d0_ref[...], d1_ref[...])
    agg = (s0_ref[...] + s1_ref[...] + p1_ref[...]) * dinv
    p2_ref[...] = (agg + b1_ref[...]) * dinv


def _stage_c_body(s0_ref, s1_ref, p2_ref, d0_ref, d1_ref,
                  w2_ref, b2_ref, w3_ref, b3_ref, m_ref, s_ref):
    dinv = _dinv_of(d0_ref[...], d1_ref[...])
    g = (s0_ref[...] + s1_ref[...] + p2_ref[...]) * dinv
    dims = (((1,), (1,)), ((), ()))
    m_ref[...] = lax.dot_general(g, w2_ref[...], dims,
                                 preferred_element_type=jnp.float32) + b2_ref[...]
    s_ref[...] = lax.dot_general(g, w3_ref[...], dims,
                                 preferred_element_type=jnp.float32) + b3_ref[...]


def _row_spec(d):
    return pl.BlockSpec((_R, d), lambda i: (i, 0))


def _full_spec(shape):
    return pl.BlockSpec(shape, lambda i: (0,) * len(shape))


def _stage_a(x, W1, d0, d1):
    return pl.pallas_call(
        _stage_a_body,
        grid=(N // _R,),
        in_specs=[_row_spec(D_IN), _full_spec(W1.shape),
                  _row_spec(16), _row_spec(16)],
        out_specs=_row_spec(D_HID),
        out_shape=jax.ShapeDtypeStruct((N, D_HID), jnp.float32),
    )(x, W1, d0, d1)


def _stage_b(s0, s1, p1, d0, d1, b1):
    return pl.pallas_call(
        _stage_b_body,
        grid=(N // _R,),
        in_specs=[_row_spec(D_HID), _row_spec(D_HID), _row_spec(D_HID),
                  _row_spec(16), _row_spec(16), _full_spec(b1.shape)],
        out_specs=_row_spec(D_HID),
        out_shape=jax.ShapeDtypeStruct((N, D_HID), jnp.float32),
    )(s0, s1, p1, d0, d1, b1)


def _stage_c(s0, s1, p2, d0, d1, W2, b2, W3, b3):
    return pl.pallas_call(
        _stage_c_body,
        grid=(N // _R,),
        in_specs=[_row_spec(D_HID), _row_spec(D_HID), _row_spec(D_HID),
                  _row_spec(16), _row_spec(16),
                  _full_spec(W2.shape), _full_spec(b2.shape),
                  _full_spec(W3.shape), _full_spec(b3.shape)],
        out_specs=[_row_spec(D_IN), _row_spec(D_IN)],
        out_shape=[jax.ShapeDtypeStruct((N, D_IN), jnp.float32),
                   jax.ShapeDtypeStruct((N, D_IN), jnp.float32)],
    )(s0, s1, p2, d0, d1, W2, b2, W3, b3)


def kernel(x, edge_index, W1, b1, W2, b2, W3, b3):
    src = edge_index[0].astype(jnp.int32)
    dst = edge_index[1].astype(jnp.int32)
    pad = E_PAD - E
    src_p = jnp.concatenate([src, jnp.zeros((pad,), jnp.int32)]).reshape(NW, NB, K)
    dst_p = jnp.concatenate([dst, jnp.full((pad,), N, jnp.int32)]).reshape(NW, NB, K)

    ones16 = jnp.ones((K, 16), jnp.float32)
    zeros16 = jnp.zeros((RPS, 16), jnp.float32)
    zeros64 = jnp.zeros((RPS, D_HID), jnp.float32)

    degp = _make_deg()(dst_p, ones16, zeros16)
    d0 = degp[0, :N]
    d1 = degp[1, :N]

    p1 = _stage_a(x, W1, d0, d1)

    s1 = _make_edge_agg(D_HID)(p1, src_p, dst_p, zeros64)
    p2 = _stage_b(s1[0, :N], s1[1, :N], p1, d0, d1, b1.reshape(1, D_HID))

    s2 = _make_edge_agg(D_HID)(p2, src_p, dst_p, zeros64)
    m, s = _stage_c(s2[0, :N], s2[1, :N], p2, d0, d1,
                    W2, b2.reshape(1, D_IN), W3, b3.reshape(1, D_IN))
    return (m, s)


# speedup vs baseline: 24.8548x; 1.0884x over previous
import functools

import jax
import jax.numpy as jnp
from jax import lax
from jax.experimental import pallas as pl
from jax.experimental.pallas import tpu as pltpu
from jax.experimental.pallas import tpu_sc as plsc

N = 10000
E = 320000
D_IN = 128
D_HID = 64

NC = 2
NS = 16
NW = NC * NS
K = 128
NB = -(-E // (NW * K))
E_PAD = NW * K * NB
N_ACC = 10112
RPS = N_ACC // NS

def _sc_mesh():
    return plsc.VectorSubcoreMesh(core_axis_name="c", subcore_axis_name="s",
                                  num_cores=NC, num_subcores=NS)


@functools.lru_cache(maxsize=None)
def _make_edge_agg(d):
    @functools.partial(
        pl.kernel,
        out_type=jax.ShapeDtypeStruct((NC, N_ACC, d), jnp.float32),
        mesh=_sc_mesh(),
        compiler_params=pltpu.CompilerParams(use_tc_tiling_on_sc=False),
        scratch_types=[
            pltpu.VMEM((NB, K), jnp.int32),
            pltpu.VMEM((NB, K), jnp.int32),
            pltpu.VMEM((K, d), jnp.float32),
            pltpu.VMEM((K, d), jnp.float32),
            pltpu.VMEM_SHARED((N_ACC, d), jnp.float32),
            pltpu.SemaphoreType.DMA,
            pltpu.SemaphoreType.DMA,
        ],
    )
    def agg(p_hbm, src_hbm, dst_hbm, zeros_hbm, out_hbm,
            src_v, dst_v, rows0_v, rows1_v, acc_sh, sem0, sem1):
        cid = lax.axis_index("c")
        sid = lax.axis_index("s")
        wid = sid * NC + cid
        pltpu.sync_copy(src_hbm.at[wid], src_v)
        pltpu.sync_copy(dst_hbm.at[wid], dst_v)
        pltpu.sync_copy(zeros_hbm, acc_sh.at[pl.ds(sid * RPS, RPS)])
        plsc.subcore_barrier()

        pltpu.async_copy(p_hbm.at[src_v.at[0]], rows0_v, sem0)

        def body(t, carry):
            j0 = 2 * t
            pltpu.make_async_copy(p_hbm.at[src_v.at[j0]], rows0_v, sem0).wait()
            pltpu.async_copy(p_hbm.at[src_v.at[j0 + 1]], rows1_v, sem1)
            pltpu.sync_copy(rows0_v, acc_sh.at[dst_v.at[j0]], add=True)
            pltpu.make_async_copy(p_hbm.at[src_v.at[j0 + 1]], rows1_v, sem1).wait()
            pltpu.async_copy(p_hbm.at[src_v.at[j0 + 2]], rows0_v, sem0)
            pltpu.sync_copy(rows1_v, acc_sh.at[dst_v.at[j0 + 1]], add=True)
            return carry

        lax.fori_loop(0, (NB - 1) // 2, body, 0)
        pltpu.make_async_copy(p_hbm.at[src_v.at[NB - 1]], rows0_v, sem0).wait()
        pltpu.sync_copy(rows0_v, acc_sh.at[dst_v.at[NB - 1]], add=True)
        plsc.subcore_barrier()
        pltpu.sync_copy(acc_sh.at[pl.ds(sid * RPS, RPS)],
                        out_hbm.at[cid, pl.ds(sid * RPS, RPS)])

    return agg


@functools.lru_cache(maxsize=None)
def _make_deg():
    @functools.partial(
        pl.kernel,
        out_type=jax.ShapeDtypeStruct((NC, N_ACC, 16), jnp.float32),
        mesh=_sc_mesh(),
        compiler_params=pltpu.CompilerParams(use_tc_tiling_on_sc=False),
        scratch_types=[
            pltpu.VMEM((NB, K), jnp.int32),
            pltpu.VMEM((K, 16), jnp.float32),
            pltpu.VMEM_SHARED((N_ACC, 16), jnp.float32),
        ],
    )
    def deg(dst_hbm, ones_hbm, zeros_hbm, out_hbm, dst_v, ones_v, acc_sh):
        cid = lax.axis_index("c")
        sid = lax.axis_index("s")
        wid = sid * NC + cid
        pltpu.sync_copy(dst_hbm.at[wid], dst_v)
        pltpu.sync_copy(ones_hbm, ones_v)
        pltpu.sync_copy(zeros_hbm, acc_sh.at[pl.ds(sid * RPS, RPS)])
        plsc.subcore_barrier()

        def body(j, carry):
            pltpu.sync_copy(ones_v, acc_sh.at[dst_v.at[j]], add=True)
            return carry

        lax.fori_loop(0, NB, body, 0)
        plsc.subcore_barrier()
        pltpu.sync_copy(acc_sh.at[pl.ds(sid * RPS, RPS)],
                        out_hbm.at[cid, pl.ds(sid * RPS, RPS)])

    return deg


_R = 1000


def _dinv_of(d0, d1):
    return lax.rsqrt(d0[:, 0:1] + d1[:, 0:1] + 1.0)


def _stage_a_body(x_ref, w1_ref, d0_ref, d1_ref, p1_ref):
    dinv = _dinv_of(d0_ref[...], d1_ref[...])
    h0 = lax.dot_general(x_ref[...], w1_ref[...], (((1,), (1,)), ((), ())),
                         preferred_element_type=jnp.float32)
    p1_ref[...] = h0 * dinv


def _stage_b_body(s0_ref, s1_ref, p1_ref, d0_ref, d1_ref, b1_ref, p2_ref):
    dinv = _dinv_of(d0_ref[...], d1_ref[...])
    agg = (s0_ref[...] + s1_ref[...] + p1_ref[...]) * dinv
    p2_ref[...] = (agg + b1_ref[...]) * dinv


def _stage_c_body(s0_ref, s1_ref, p2_ref, d0_ref, d1_ref,
                  w2_ref, b2_ref, w3_ref, b3_ref, m_ref, s_ref):
    dinv = _dinv_of(d0_ref[...], d1_ref[...])
    g = (s0_ref[...] + s1_ref[...] + p2_ref[...]) * dinv
    dims = (((1,), (1,)), ((), ()))
    m_ref[...] = lax.dot_general(g, w2_ref[...], dims,
                                 preferred_element_type=jnp.float32) + b2_ref[...]
    s_ref[...] = lax.dot_general(g, w3_ref[...], dims,
                                 preferred_element_type=jnp.float32) + b3_ref[...]


def _row_spec(d):
    return pl.BlockSpec((_R, d), lambda i: (i, 0))


def _full_spec(shape):
    return pl.BlockSpec(shape, lambda i: (0,) * len(shape))


def _stage_a(x, W1, d0, d1):
    return pl.pallas_call(
        _stage_a_body,
        grid=(N // _R,),
        in_specs=[_row_spec(D_IN), _full_spec(W1.shape),
                  _row_spec(16), _row_spec(16)],
        out_specs=_row_spec(D_HID),
        out_shape=jax.ShapeDtypeStruct((N, D_HID), jnp.float32),
    )(x, W1, d0, d1)


def _stage_b(s0, s1, p1, d0, d1, b1):
    return pl.pallas_call(
        _stage_b_body,
        grid=(N // _R,),
        in_specs=[_row_spec(D_HID), _row_spec(D_HID), _row_spec(D_HID),
                  _row_spec(16), _row_spec(16), _full_spec(b1.shape)],
        out_specs=_row_spec(D_HID),
        out_shape=jax.ShapeDtypeStruct((N, D_HID), jnp.float32),
    )(s0, s1, p1, d0, d1, b1)


def _stage_c(s0, s1, p2, d0, d1, W2, b2, W3, b3):
    return pl.pallas_call(
        _stage_c_body,
        grid=(N // _R,),
        in_specs=[_row_spec(D_HID), _row_spec(D_HID), _row_spec(D_HID),
                  _row_spec(16), _row_spec(16),
                  _full_spec(W2.shape), _full_spec(b2.shape),
                  _full_spec(W3.shape), _full_spec(b3.shape)],
        out_specs=[_row_spec(D_IN), _row_spec(D_IN)],
        out_shape=[jax.ShapeDtypeStruct((N, D_IN), jnp.float32),
                   jax.ShapeDtypeStruct((N, D_IN), jnp.float32)],
    )(s0, s1, p2, d0, d1, W2, b2, W3, b3)


def kernel(x, edge_index, W1, b1, W2, b2, W3, b3):
    src = edge_index[0].astype(jnp.int32)
    dst = edge_index[1].astype(jnp.int32)
    pad = E_PAD - E
    src_p = jnp.concatenate([src, jnp.zeros((pad,), jnp.int32)]).reshape(NW, NB, K)
    dst_p = jnp.concatenate([dst, jnp.full((pad,), N, jnp.int32)]).reshape(NW, NB, K)

    ones16 = jnp.ones((K, 16), jnp.float32)
    zeros16 = jnp.zeros((RPS, 16), jnp.float32)
    zeros64 = jnp.zeros((RPS, D_HID), jnp.float32)

    degp = _make_deg()(dst_p, ones16, zeros16)
    d0 = degp[0, :N]
    d1 = degp[1, :N]

    p1 = _stage_a(x, W1, d0, d1)

    s1 = _make_edge_agg(D_HID)(p1, src_p, dst_p, zeros64)
    p2 = _stage_b(s1[0, :N], s1[1, :N], p1, d0, d1, b1.reshape(1, D_HID))

    s2 = _make_edge_agg(D_HID)(p2, src_p, dst_p, zeros64)
    m, s = _stage_c(s2[0, :N], s2[1, :N], p2, d0, d1,
                    W2, b2.reshape(1, D_IN), W3, b3.reshape(1, D_IN))
    return (m, s)


# speedup vs baseline: 25.6855x; 1.0334x over previous
import functools

import jax
import jax.numpy as jnp
from jax import lax
from jax.experimental import pallas as pl
from jax.experimental.pallas import tpu as pltpu
from jax.experimental.pallas import tpu_sc as plsc

N = 10000
E = 320000
D_IN = 128
D_HID = 64

NC = 2
NS = 16
NW = NC * NS
K = 128
NB = -(-E // (NW * K))
E_PAD = NW * K * NB
N_ACC = 10112
RPS = N_ACC // NS

def _sc_mesh():
    return plsc.VectorSubcoreMesh(core_axis_name="c", subcore_axis_name="s",
                                  num_cores=NC, num_subcores=NS)


@functools.lru_cache(maxsize=None)
def _make_edge_agg(d):
    @functools.partial(
        pl.kernel,
        out_type=jax.ShapeDtypeStruct((NC, N_ACC, d), jnp.float32),
        mesh=_sc_mesh(),
        compiler_params=pltpu.CompilerParams(use_tc_tiling_on_sc=False),
        scratch_types=[
            pltpu.VMEM((NB, K), jnp.int32),
            pltpu.VMEM((NB, K), jnp.int32),
            pltpu.VMEM((K, d), jnp.float32),
            pltpu.VMEM((K, d), jnp.float32),
            pltpu.VMEM_SHARED((N_ACC, d), jnp.float32),
            pltpu.SemaphoreType.DMA,
            pltpu.SemaphoreType.DMA,
        ],
    )
    def agg(p_hbm, src_hbm, dst_hbm, zeros_hbm, out_hbm,
            src_v, dst_v, rows0_v, rows1_v, acc_sh, sem0, sem1):
        cid = lax.axis_index("c")
        sid = lax.axis_index("s")
        wid = sid * NC + cid
        tab = p_hbm.at[cid]
        pltpu.sync_copy(src_hbm.at[wid], src_v)
        pltpu.sync_copy(dst_hbm.at[wid], dst_v)
        pltpu.sync_copy(zeros_hbm, acc_sh.at[pl.ds(sid * RPS, RPS)])
        plsc.subcore_barrier()

        pltpu.async_copy(tab.at[src_v.at[0]], rows0_v, sem0)

        def body(t, carry):
            j0 = 2 * t
            pltpu.make_async_copy(tab.at[src_v.at[j0]], rows0_v, sem0).wait()
            pltpu.async_copy(tab.at[src_v.at[j0 + 1]], rows1_v, sem1)
            pltpu.sync_copy(rows0_v, acc_sh.at[dst_v.at[j0]], add=True)
            pltpu.make_async_copy(tab.at[src_v.at[j0 + 1]], rows1_v, sem1).wait()
            pltpu.async_copy(tab.at[src_v.at[j0 + 2]], rows0_v, sem0)
            pltpu.sync_copy(rows1_v, acc_sh.at[dst_v.at[j0 + 1]], add=True)
            return carry

        lax.fori_loop(0, (NB - 1) // 2, body, 0)
        pltpu.make_async_copy(tab.at[src_v.at[NB - 1]], rows0_v, sem0).wait()
        pltpu.sync_copy(rows0_v, acc_sh.at[dst_v.at[NB - 1]], add=True)
        plsc.subcore_barrier()
        pltpu.sync_copy(acc_sh.at[pl.ds(sid * RPS, RPS)],
                        out_hbm.at[cid, pl.ds(sid * RPS, RPS)])

    return agg


@functools.lru_cache(maxsize=None)
def _make_deg():
    @functools.partial(
        pl.kernel,
        out_type=jax.ShapeDtypeStruct((NC, N_ACC, 16), jnp.float32),
        mesh=_sc_mesh(),
        compiler_params=pltpu.CompilerParams(use_tc_tiling_on_sc=False),
        scratch_types=[
            pltpu.VMEM((NB, K), jnp.int32),
            pltpu.VMEM((K, 16), jnp.float32),
            pltpu.VMEM_SHARED((N_ACC, 16), jnp.float32),
        ],
    )
    def deg(dst_hbm, ones_hbm, zeros_hbm, out_hbm, dst_v, ones_v, acc_sh):
        cid = lax.axis_index("c")
        sid = lax.axis_index("s")
        wid = sid * NC + cid
        pltpu.sync_copy(dst_hbm.at[wid], dst_v)
        pltpu.sync_copy(ones_hbm, ones_v)
        pltpu.sync_copy(zeros_hbm, acc_sh.at[pl.ds(sid * RPS, RPS)])
        plsc.subcore_barrier()

        def body(j, carry):
            pltpu.sync_copy(ones_v, acc_sh.at[dst_v.at[j]], add=True)
            return carry

        lax.fori_loop(0, NB, body, 0)
        plsc.subcore_barrier()
        pltpu.sync_copy(acc_sh.at[pl.ds(sid * RPS, RPS)],
                        out_hbm.at[cid, pl.ds(sid * RPS, RPS)])

    return deg


_R = 1000


def _dinv_of(d0, d1):
    return lax.rsqrt(d0[:, 0:1] + d1[:, 0:1] + 1.0)


def _stage_a_body(x_ref, w1_ref, d0_ref, d1_ref, p1_ref):
    dinv = _dinv_of(d0_ref[0], d1_ref[0])
    h0 = lax.dot_general(x_ref[...], w1_ref[...], (((1,), (1,)), ((), ())),
                         preferred_element_type=jnp.float32)
    v = h0 * dinv
    p1_ref[0] = v
    p1_ref[1] = v


def _stage_b_body(s0_ref, s1_ref, p1_ref, d0_ref, d1_ref, b1_ref, p2_ref):
    dinv = _dinv_of(d0_ref[0], d1_ref[0])
    agg = (s0_ref[0] + s1_ref[0] + p1_ref[0]) * dinv
    v = (agg + b1_ref[...]) * dinv
    p2_ref[0] = v
    p2_ref[1] = v


def _stage_c_body(s0_ref, s1_ref, p2_ref, d0_ref, d1_ref,
                  w2_ref, b2_ref, w3_ref, b3_ref, m_ref, s_ref):
    dinv = _dinv_of(d0_ref[0], d1_ref[0])
    g = (s0_ref[0] + s1_ref[0] + p2_ref[0]) * dinv
    dims = (((1,), (1,)), ((), ()))
    m_ref[...] = lax.dot_general(g, w2_ref[...], dims,
                                 preferred_element_type=jnp.float32) + b2_ref[...]
    s_ref[...] = lax.dot_general(g, w3_ref[...], dims,
                                 preferred_element_type=jnp.float32) + b3_ref[...]


def _row_spec(d):
    return pl.BlockSpec((_R, d), lambda i: (i, 0))


def _part_spec(c, d):
    return pl.BlockSpec((1, _R, d), lambda i, _c=c: (_c, i, 0))


def _full_spec(shape):
    return pl.BlockSpec(shape, lambda i: (0,) * len(shape))


def _stage_a(x, W1, degp):
    return pl.pallas_call(
        _stage_a_body,
        grid=(N // _R,),
        in_specs=[_row_spec(D_IN), _full_spec(W1.shape),
                  _part_spec(0, 16), _part_spec(1, 16)],
        out_specs=pl.BlockSpec((NC, _R, D_HID), lambda i: (0, i, 0)),
        out_shape=jax.ShapeDtypeStruct((NC, N, D_HID), jnp.float32),
    )(x, W1, degp, degp)


def _stage_b(s1, p1, degp, b1):
    return pl.pallas_call(
        _stage_b_body,
        grid=(N // _R,),
        in_specs=[_part_spec(0, D_HID), _part_spec(1, D_HID),
                  _part_spec(0, D_HID),
                  _part_spec(0, 16), _part_spec(1, 16), _full_spec(b1.shape)],
        out_specs=pl.BlockSpec((NC, _R, D_HID), lambda i: (0, i, 0)),
        out_shape=jax.ShapeDtypeStruct((NC, N, D_HID), jnp.float32),
    )(s1, s1, p1, degp, degp, b1)


def _stage_c(s2, p2, degp, W2, b2, W3, b3):
    return pl.pallas_call(
        _stage_c_body,
        grid=(N // _R,),
        in_specs=[_part_spec(0, D_HID), _part_spec(1, D_HID),
                  _part_spec(0, D_HID),
                  _part_spec(0, 16), _part_spec(1, 16),
                  _full_spec(W2.shape), _full_spec(b2.shape),
                  _full_spec(W3.shape), _full_spec(b3.shape)],
        out_specs=[_row_spec(D_IN), _row_spec(D_IN)],
        out_shape=[jax.ShapeDtypeStruct((N, D_IN), jnp.float32),
                   jax.ShapeDtypeStruct((N, D_IN), jnp.float32)],
    )(s2, s2, p2, degp, degp, W2, b2, W3, b3)


def kernel(x, edge_index, W1, b1, W2, b2, W3, b3):
    src = edge_index[0].astype(jnp.int32)
    dst = edge_index[1].astype(jnp.int32)
    pad = E_PAD - E
    src_p = jnp.concatenate([src, jnp.zeros((pad,), jnp.int32)]).reshape(NW, NB, K)
    dst_p = jnp.concatenate([dst, jnp.full((pad,), N, jnp.int32)]).reshape(NW, NB, K)

    ones16 = jnp.ones((K, 16), jnp.float32)
    zeros16 = jnp.zeros((RPS, 16), jnp.float32)
    zeros64 = jnp.zeros((RPS, D_HID), jnp.float32)

    degp = _make_deg()(dst_p, ones16, zeros16)

    p1 = _stage_a(x, W1, degp)
    s1 = _make_edge_agg(D_HID)(p1, src_p, dst_p, zeros64)

    p2 = _stage_b(s1, p1, degp, b1.reshape(1, D_HID))
    s2 = _make_edge_agg(D_HID)(p2, src_p, dst_p, zeros64)

    m, s = _stage_c(s2, p2, degp,
                    W2, b2.reshape(1, D_IN), W3, b3.reshape(1, D_IN))
    return (m, s)


# speedup vs baseline: 29.0643x; 1.1315x over previous
import functools

import jax
import jax.numpy as jnp
from jax import lax
from jax.experimental import pallas as pl
from jax.experimental.pallas import tpu as pltpu
from jax.experimental.pallas import tpu_sc as plsc

N = 10000
E = 320000
D_IN = 128
D_HID = 64

NC = 2
NS = 16
NW = NC * NS
K = 128
NB = -(-E // (NW * K))
E_PAD = NW * K * NB
N_ACC = 10112
RPS = N_ACC // NS

def _sc_mesh():
    return plsc.VectorSubcoreMesh(core_axis_name="c", subcore_axis_name="s",
                                  num_cores=NC, num_subcores=NS)


@functools.lru_cache(maxsize=None)
def _make_edge_agg(d):
    NBUF = 8
    PF = 4

    @functools.partial(
        pl.kernel,
        out_type=jax.ShapeDtypeStruct((NC, N_ACC, d), jnp.float32),
        mesh=_sc_mesh(),
        compiler_params=pltpu.CompilerParams(use_tc_tiling_on_sc=False),
        scratch_types=(
            [pltpu.VMEM((NB, K), jnp.int32)] * 2
            + [pltpu.VMEM((K, d), jnp.float32)] * NBUF
            + [pltpu.VMEM_SHARED((N_ACC, d), jnp.float32)]
            + [pltpu.SemaphoreType.DMA] * (2 * NBUF)
        ),
    )
    def agg(p_hbm, src_hbm, dst_hbm, zeros_hbm, out_hbm, src_v, dst_v, *rest):
        rows = rest[0:NBUF]
        acc_sh = rest[NBUF]
        gsem = rest[NBUF + 1:2 * NBUF + 1]
        ssem = rest[2 * NBUF + 1:3 * NBUF + 1]
        cid = lax.axis_index("c")
        sid = lax.axis_index("s")
        wid = sid * NC + cid
        tab = p_hbm.at[cid]
        pltpu.sync_copy(src_hbm.at[wid], src_v)
        pltpu.sync_copy(dst_hbm.at[wid], dst_v)
        pltpu.sync_copy(zeros_hbm, acc_sh.at[pl.ds(sid * RPS, RPS)])
        plsc.subcore_barrier()

        def gather(j, b):
            pltpu.async_copy(tab.at[src_v.at[j]], rows[b], gsem[b])

        def gwait(j, b):
            pltpu.make_async_copy(tab.at[src_v.at[j]], rows[b], gsem[b]).wait()

        def scat(j, b):
            pltpu.async_copy(rows[b], acc_sh.at[dst_v.at[j]], ssem[b], add=True)

        def swait(j, b):
            pltpu.make_async_copy(rows[b], acc_sh.at[dst_v.at[j]],
                                  ssem[b]).wait()

        for j in range(PF):
            gather(j, j % NBUF)
        for j in range(NBUF):
            if j + PF < NB:
                bp = (j + PF) % NBUF
                if j + PF >= NBUF:
                    swait(j + PF - NBUF, bp)
                gather(j + PF, bp)
            gwait(j, j % NBUF)
            scat(j, j % NBUF)

        def body(t, carry):
            for b in range(NBUF):
                j = NBUF * t + b
                bp = (b + PF) % NBUF
                swait(j + PF - NBUF, bp)
                gather(j + PF, bp)
                gwait(j, b)
                scat(j, b)
            return carry

        n_main = (NB - PF) // NBUF
        lax.fori_loop(1, n_main, body, 0)
        for j in range(NBUF * n_main, NB):
            b = j % NBUF
            if j + PF < NB:
                bp = (j + PF) % NBUF
                swait(j + PF - NBUF, bp)
                gather(j + PF, bp)
            gwait(j, b)
            scat(j, b)
        for j in range(NB - NBUF, NB):
            swait(j, j % NBUF)
        plsc.subcore_barrier()
        pltpu.sync_copy(acc_sh.at[pl.ds(sid * RPS, RPS)],
                        out_hbm.at[cid, pl.ds(sid * RPS, RPS)])

    return agg


@functools.lru_cache(maxsize=None)
def _make_deg():
    @functools.partial(
        pl.kernel,
        out_type=jax.ShapeDtypeStruct((NC, N_ACC, 16), jnp.float32),
        mesh=_sc_mesh(),
        compiler_params=pltpu.CompilerParams(use_tc_tiling_on_sc=False),
        scratch_types=[
            pltpu.VMEM((NB, K), jnp.int32),
            pltpu.VMEM((K, 16), jnp.float32),
            pltpu.VMEM_SHARED((N_ACC, 16), jnp.float32),
        ],
    )
    def deg(dst_hbm, ones_hbm, zeros_hbm, out_hbm, dst_v, ones_v, acc_sh):
        cid = lax.axis_index("c")
        sid = lax.axis_index("s")
        wid = sid * NC + cid
        pltpu.sync_copy(dst_hbm.at[wid], dst_v)
        pltpu.sync_copy(ones_hbm, ones_v)
        pltpu.sync_copy(zeros_hbm, acc_sh.at[pl.ds(sid * RPS, RPS)])
        plsc.subcore_barrier()

        def body(j, carry):
            pltpu.sync_copy(ones_v, acc_sh.at[dst_v.at[j]], add=True)
            return carry

        lax.fori_loop(0, NB, body, 0)
        plsc.subcore_barrier()
        pltpu.sync_copy(acc_sh.at[pl.ds(sid * RPS, RPS)],
                        out_hbm.at[cid, pl.ds(sid * RPS, RPS)])

    return deg


_R = 1000


def _dinv_of(d0, d1):
    return lax.rsqrt(d0[:, 0:1] + d1[:, 0:1] + 1.0)


def _stage_a_body(x_ref, w1_ref, d0_ref, d1_ref, p1_ref):
    dinv = _dinv_of(d0_ref[0], d1_ref[0])
    h0 = lax.dot_general(x_ref[...], w1_ref[...], (((1,), (1,)), ((), ())),
                         preferred_element_type=jnp.float32)
    v = h0 * dinv
    p1_ref[0] = v
    p1_ref[1] = v


def _stage_b_body(s0_ref, s1_ref, p1_ref, d0_ref, d1_ref, b1_ref, p2_ref):
    dinv = _dinv_of(d0_ref[0], d1_ref[0])
    agg = (s0_ref[0] + s1_ref[0] + p1_ref[0]) * dinv
    v = (agg + b1_ref[...]) * dinv
    p2_ref[0] = v
    p2_ref[1] = v


def _stage_c_body(s0_ref, s1_ref, p2_ref, d0_ref, d1_ref,
                  w2_ref, b2_ref, w3_ref, b3_ref, m_ref, s_ref):
    dinv = _dinv_of(d0_ref[0], d1_ref[0])
    g = (s0_ref[0] + s1_ref[0] + p2_ref[0]) * dinv
    dims = (((1,), (1,)), ((), ()))
    m_ref[...] = lax.dot_general(g, w2_ref[...], dims,
                                 preferred_element_type=jnp.float32) + b2_ref[...]
    s_ref[...] = lax.dot_general(g, w3_ref[...], dims,
                                 preferred_element_type=jnp.float32) + b3_ref[...]


def _row_spec(d):
    return pl.BlockSpec((_R, d), lambda i: (i, 0))


def _part_spec(c, d):
    return pl.BlockSpec((1, _R, d), lambda i, _c=c: (_c, i, 0))


def _full_spec(shape):
    return pl.BlockSpec(shape, lambda i: (0,) * len(shape))


def _stage_a(x, W1, degp):
    return pl.pallas_call(
        _stage_a_body,
        grid=(N // _R,),
        in_specs=[_row_spec(D_IN), _full_spec(W1.shape),
                  _part_spec(0, 16), _part_spec(1, 16)],
        out_specs=pl.BlockSpec((NC, _R, D_HID), lambda i: (0, i, 0)),
        out_shape=jax.ShapeDtypeStruct((NC, N, D_HID), jnp.float32),
    )(x, W1, degp, degp)


def _stage_b(s1, p1, degp, b1):
    return pl.pallas_call(
        _stage_b_body,
        grid=(N // _R,),
        in_specs=[_part_spec(0, D_HID), _part_spec(1, D_HID),
                  _part_spec(0, D_HID),
                  _part_spec(0, 16), _part_spec(1, 16), _full_spec(b1.shape)],
        out_specs=pl.BlockSpec((NC, _R, D_HID), lambda i: (0, i, 0)),
        out_shape=jax.ShapeDtypeStruct((NC, N, D_HID), jnp.float32),
    )(s1, s1, p1, degp, degp, b1)


def _stage_c(s2, p2, degp, W2, b2, W3, b3):
    return pl.pallas_call(
        _stage_c_body,
        grid=(N // _R,),
        in_specs=[_part_spec(0, D_HID), _part_spec(1, D_HID),
                  _part_spec(0, D_HID),
                  _part_spec(0, 16), _part_spec(1, 16),
                  _full_spec(W2.shape), _full_spec(b2.shape),
                  _full_spec(W3.shape), _full_spec(b3.shape)],
        out_specs=[_row_spec(D_IN), _row_spec(D_IN)],
        out_shape=[jax.ShapeDtypeStruct((N, D_IN), jnp.float32),
                   jax.ShapeDtypeStruct((N, D_IN), jnp.float32)],
    )(s2, s2, p2, degp, degp, W2, b2, W3, b3)


def kernel(x, edge_index, W1, b1, W2, b2, W3, b3):
    src = edge_index[0].astype(jnp.int32)
    dst = edge_index[1].astype(jnp.int32)
    pad = E_PAD - E
    src_p = jnp.concatenate([src, jnp.zeros((pad,), jnp.int32)]).reshape(NW, NB, K)
    dst_p = jnp.concatenate([dst, jnp.full((pad,), N, jnp.int32)]).reshape(NW, NB, K)

    ones16 = jnp.ones((K, 16), jnp.float32)
    zeros16 = jnp.zeros((RPS, 16), jnp.float32)
    zeros64 = jnp.zeros((RPS, D_HID), jnp.float32)

    degp = _make_deg()(dst_p, ones16, zeros16)

    p1 = _stage_a(x, W1, degp)
    s1 = _make_edge_agg(D_HID)(p1, src_p, dst_p, zeros64)

    p2 = _stage_b(s1, p1, degp, b1.reshape(1, D_HID))
    s2 = _make_edge_agg(D_HID)(p2, src_p, dst_p, zeros64)

    m, s = _stage_c(s2, p2, degp,
                    W2, b2.reshape(1, D_IN), W3, b3.reshape(1, D_IN))
    return (m, s)


# speedup vs baseline: 45.8945x; 1.5791x over previous
import functools

import jax
import jax.numpy as jnp
from jax import lax
from jax.experimental import pallas as pl
from jax.experimental.pallas import tpu as pltpu
from jax.experimental.pallas import tpu_sc as plsc

N = 10000
E = 320000
D_IN = 128
D_HID = 64
D_HALF = D_HID // 2

NC = 2
NS = 16
NW = NC * NS
K = 128
NB1 = -(-E // (NW * K))
E_PAD1 = NW * K * NB1
NB2 = -(-E // (NS * K))
E_PAD2 = NS * K * NB2
N_ACC = 10112
RPS = N_ACC // NS

def _sc_mesh():
    return plsc.VectorSubcoreMesh(core_axis_name="c", subcore_axis_name="s",
                                  num_cores=NC, num_subcores=NS)


@functools.lru_cache(maxsize=None)
def _make_edge_agg():
    NBUF = 8
    PF = 4

    @functools.partial(
        pl.kernel,
        out_type=jax.ShapeDtypeStruct((NC, N_ACC, D_HALF), jnp.float32),
        mesh=_sc_mesh(),
        compiler_params=pltpu.CompilerParams(use_tc_tiling_on_sc=False),
        scratch_types=(
            [pltpu.VMEM((NB2, K), jnp.int32)] * 2
            + [pltpu.VMEM((K, D_HALF), jnp.float32)] * NBUF
            + [pltpu.VMEM_SHARED((N_ACC, D_HALF), jnp.float32)] * 2
            + [pltpu.SemaphoreType.DMA] * (2 * NBUF)
        ),
    )
    def agg(p_hbm, src_hbm, dst_hbm, zeros_hbm, out_hbm, src_v, dst_v, *rest):
        rows = rest[0:NBUF]
        acc_sh = rest[NBUF]
        tab = rest[NBUF + 1]
        gsem = rest[NBUF + 2:2 * NBUF + 2]
        ssem = rest[2 * NBUF + 2:3 * NBUF + 2]
        cid = lax.axis_index("c")
        sid = lax.axis_index("s")
        pltpu.sync_copy(src_hbm.at[sid], src_v)
        pltpu.sync_copy(dst_hbm.at[sid], dst_v)
        pltpu.sync_copy(p_hbm.at[cid, pl.ds(sid * RPS, RPS)],
                        tab.at[pl.ds(sid * RPS, RPS)])
        pltpu.sync_copy(zeros_hbm, acc_sh.at[pl.ds(sid * RPS, RPS)])
        plsc.subcore_barrier()

        def gather(j, b):
            pltpu.async_copy(tab.at[src_v.at[j]], rows[b], gsem[b])

        def gwait(j, b):
            pltpu.make_async_copy(tab.at[src_v.at[j]], rows[b], gsem[b]).wait()

        def scat(j, b):
            pltpu.async_copy(rows[b], acc_sh.at[dst_v.at[j]], ssem[b], add=True)

        def swait(j, b):
            pltpu.make_async_copy(rows[b], acc_sh.at[dst_v.at[j]],
                                  ssem[b]).wait()

        for j in range(PF):
            gather(j, j % NBUF)
        for j in range(NBUF):
            if j + PF < NB2:
                bp = (j + PF) % NBUF
                if j + PF >= NBUF:
                    swait(j + PF - NBUF, bp)
                gather(j + PF, bp)
            gwait(j, j % NBUF)
            scat(j, j % NBUF)

        def body(t, carry):
            for b in range(NBUF):
                j = NBUF * t + b
                bp = (b + PF) % NBUF
                swait(j + PF - NBUF, bp)
                gather(j + PF, bp)
                gwait(j, b)
                scat(j, b)
            return carry

        n_main = (NB2 - PF) // NBUF
        lax.fori_loop(1, n_main, body, 0)
        for j in range(NBUF * n_main, NB2):
            b = j % NBUF
            if j + PF < NB2:
                bp = (j + PF) % NBUF
                swait(j + PF - NBUF, bp)
                gather(j + PF, bp)
            gwait(j, b)
            scat(j, b)
        for j in range(NB2 - NBUF, NB2):
            swait(j, j % NBUF)
        plsc.subcore_barrier()
        pltpu.sync_copy(acc_sh.at[pl.ds(sid * RPS, RPS)],
                        out_hbm.at[cid, pl.ds(sid * RPS, RPS)])

    return agg


@functools.lru_cache(maxsize=None)
def _make_deg():
    @functools.partial(
        pl.kernel,
        out_type=jax.ShapeDtypeStruct((NC, N_ACC, 16), jnp.float32),
        mesh=_sc_mesh(),
        compiler_params=pltpu.CompilerParams(use_tc_tiling_on_sc=False),
        scratch_types=[
            pltpu.VMEM((NB1, K), jnp.int32),
            pltpu.VMEM((K, 16), jnp.float32),
            pltpu.VMEM_SHARED((N_ACC, 16), jnp.float32),
        ],
    )
    def deg(dst_hbm, ones_hbm, zeros_hbm, out_hbm, dst_v, ones_v, acc_sh):
        cid = lax.axis_index("c")
        sid = lax.axis_index("s")
        wid = sid * NC + cid
        pltpu.sync_copy(dst_hbm.at[wid], dst_v)
        pltpu.sync_copy(ones_hbm, ones_v)
        pltpu.sync_copy(zeros_hbm, acc_sh.at[pl.ds(sid * RPS, RPS)])
        plsc.subcore_barrier()

        def body(j, carry):
            pltpu.sync_copy(ones_v, acc_sh.at[dst_v.at[j]], add=True)
            return carry

        lax.fori_loop(0, NB1, body, 0)
        plsc.subcore_barrier()
        pltpu.sync_copy(acc_sh.at[pl.ds(sid * RPS, RPS)],
                        out_hbm.at[cid, pl.ds(sid * RPS, RPS)])

    return deg


_R = 1000


def _dinv_of(d0, d1):
    return lax.rsqrt(d0[:, 0:1] + d1[:, 0:1] + 1.0)


def _cat(a_ref, b_ref):
    return jnp.concatenate([a_ref[0], b_ref[0]], axis=-1)


def _stage_a_body(x_ref, w1_ref, d0_ref, d1_ref, p1_ref):
    dinv = _dinv_of(d0_ref[0], d1_ref[0])
    h0 = lax.dot_general(x_ref[...], w1_ref[...], (((1,), (1,)), ((), ())),
                         preferred_element_type=jnp.float32)
    v = h0 * dinv
    p1_ref[0] = v[:, :D_HALF]
    p1_ref[1] = v[:, D_HALF:]


def _stage_b_body(s0_ref, s1_ref, pa_ref, pb_ref, d0_ref, d1_ref, b1_ref,
                  p2_ref):
    dinv = _dinv_of(d0_ref[0], d1_ref[0])
    agg = (_cat(s0_ref, s1_ref) + _cat(pa_ref, pb_ref)) * dinv
    v = (agg + b1_ref[...]) * dinv
    p2_ref[0] = v[:, :D_HALF]
    p2_ref[1] = v[:, D_HALF:]


def _stage_c_body(s0_ref, s1_ref, pa_ref, pb_ref, d0_ref, d1_ref,
                  w2_ref, b2_ref, w3_ref, b3_ref, m_ref, s_ref):
    dinv = _dinv_of(d0_ref[0], d1_ref[0])
    g = (_cat(s0_ref, s1_ref) + _cat(pa_ref, pb_ref)) * dinv
    dims = (((1,), (1,)), ((), ()))
    m_ref[...] = lax.dot_general(g, w2_ref[...], dims,
                                 preferred_element_type=jnp.float32) + b2_ref[...]
    s_ref[...] = lax.dot_general(g, w3_ref[...], dims,
                                 preferred_element_type=jnp.float32) + b3_ref[...]


def _row_spec(d):
    return pl.BlockSpec((_R, d), lambda i: (i, 0))


def _part_spec(c, d):
    return pl.BlockSpec((1, _R, d), lambda i, _c=c: (_c, i, 0))


def _full_spec(shape):
    return pl.BlockSpec(shape, lambda i: (0,) * len(shape))


def _split_out_spec():
    return pl.BlockSpec((NC, _R, D_HALF), lambda i: (0, i, 0))


def _stage_a(x, W1, degp):
    return pl.pallas_call(
        _stage_a_body,
        grid=(N // _R,),
        in_specs=[_row_spec(D_IN), _full_spec(W1.shape),
                  _part_spec(0, 16), _part_spec(1, 16)],
        out_specs=_split_out_spec(),
        out_shape=jax.ShapeDtypeStruct((NC, N_ACC, D_HALF), jnp.float32),
    )(x, W1, degp, degp)


def _stage_b(s1, p1, degp, b1):
    return pl.pallas_call(
        _stage_b_body,
        grid=(N // _R,),
        in_specs=[_part_spec(0, D_HALF), _part_spec(1, D_HALF),
                  _part_spec(0, D_HALF), _part_spec(1, D_HALF),
                  _part_spec(0, 16), _part_spec(1, 16), _full_spec(b1.shape)],
        out_specs=_split_out_spec(),
        out_shape=jax.ShapeDtypeStruct((NC, N_ACC, D_HALF), jnp.float32),
    )(s1, s1, p1, p1, degp, degp, b1)


def _stage_c(s2, p2, degp, W2, b2, W3, b3):
    return pl.pallas_call(
        _stage_c_body,
        grid=(N // _R,),
        in_specs=[_part_spec(0, D_HALF), _part_spec(1, D_HALF),
                  _part_spec(0, D_HALF), _part_spec(1, D_HALF),
                  _part_spec(0, 16), _part_spec(1, 16),
                  _full_spec(W2.shape), _full_spec(b2.shape),
                  _full_spec(W3.shape), _full_spec(b3.shape)],
        out_specs=[_row_spec(D_IN), _row_spec(D_IN)],
        out_shape=[jax.ShapeDtypeStruct((N, D_IN), jnp.float32),
                   jax.ShapeDtypeStruct((N, D_IN), jnp.float32)],
    )(s2, s2, p2, p2, degp, degp, W2, b2, W3, b3)


def kernel(x, edge_index, W1, b1, W2, b2, W3, b3):
    src = edge_index[0].astype(jnp.int32)
    dst = edge_index[1].astype(jnp.int32)
    dst_deg = jnp.concatenate(
        [dst, jnp.full((E_PAD1 - E,), N, jnp.int32)]).reshape(NW, NB1, K)
    src_agg = jnp.concatenate(
        [src, jnp.zeros((E_PAD2 - E,), jnp.int32)]).reshape(NS, NB2, K)
    dst_agg = jnp.concatenate(
        [dst, jnp.full((E_PAD2 - E,), N, jnp.int32)]).reshape(NS, NB2, K)

    ones16 = jnp.ones((K, 16), jnp.float32)
    zeros16 = jnp.zeros((RPS, 16), jnp.float32)
    zeros32 = jnp.zeros((RPS, D_HALF), jnp.float32)

    degp = _make_deg()(dst_deg, ones16, zeros16)

    p1 = _stage_a(x, W1, degp)
    s1 = _make_edge_agg()(p1, src_agg, dst_agg, zeros32)

    p2 = _stage_b(s1, p1, degp, b1.reshape(1, D_HID))
    s2 = _make_edge_agg()(p2, src_agg, dst_agg, zeros32)

    m, s = _stage_c(s2, p2, degp,
                    W2, b2.reshape(1, D_IN), W3, b3.reshape(1, D_IN))
    return (m, s)
